# Initial kernel scaffold; baseline (speedup 1.0000x reference)
import jax
import jax.numpy as jnp
from jax.experimental import pallas as pl


def kernel(x, pos, edge_index, edge_attr, batch, Wc0, bc0, We0, be0, Wx0, bx0, Wh0, bh0, Wc1, bc1, We1, be1, Wx1, bx1, Wh1, bh1, Wo1, bo1, Wo2, bo2):
    raise NotImplementedError("write your pallas kernel here")


# speedup vs baseline: 2.4494x; 2.4494x over previous
import functools

import jax
import jax.numpy as jnp
from jax import lax
from jax.experimental import pallas as pl
from jax.experimental.pallas import tpu as pltpu
from jax.experimental.pallas import tpu_sc as plsc

N = 10000
E = 160000
D = 128
H = 128
EC = 16
G = 64

NC = 2
NS = 16
NW = NC * NS
L = 16

B = 40
EPW = E // NW
KPW = EPW // B
GC = 5
KG = KPW // GC
N_PAD = 10240
ROWS_PER_TILE = N_PAD // NS


def _sc_edge_body(with_pos, *refs):
    if with_pos:
        (src3d, dst3d, xs_hbm, xdb_hbm, ea_hbm, pos_hbm,
         v_hbm, wx_hbm, bx_hbm, iota_hbm,
         magg_out, pd_out, pde_hbm,
         sidx, didx, xs_blk, xd_blk, eam_blk, pblk, ps_blk, pd_blk,
         v_vm, wx_vm, bx_vm, idxz, magg_acc) = refs
    else:
        (src3d, dst3d, xs_hbm, xdb_hbm, ea_hbm, pos_hbm,
         v_hbm, iota_hbm,
         magg_out,
         sidx, didx, xs_blk, xd_blk, eam_blk, ps_blk, pd_blk,
         v_vm, idxz, magg_acc) = refs

    c = lax.axis_index("c")
    s = lax.axis_index("s")
    w = s * NC + c

    pltpu.sync_copy(v_hbm, v_vm)
    if with_pos:
        pltpu.sync_copy(wx_hbm, wx_vm)
        pltpu.sync_copy(bx_hbm, bx_vm)

    zero16 = jnp.zeros((L,), jnp.float32)

    def _zero_row(i, _):
        for j in range(H // L):
            eam_blk[i, pl.ds(j * L, L)] = zero16
        return 0

    lax.fori_loop(0, B, _zero_row, 0)
    pltpu.sync_copy(iota_hbm.at[s], idxz)
    for t in range(ROWS_PER_TILE // B):
        pltpu.sync_copy(eam_blk, magg_acc.at[idxz.at[t]])
    plsc.subcore_barrier()

    v_regs = [v_vm[pl.ds(j * L, L)] for j in range(H // L)]
    if with_pos:
        wx_regs = [wx_vm[pl.ds(j * L, L)] for j in range(H // L)]
        bx_s = bx_vm[...][0]
        io = lax.iota(jnp.int32, L)
        degv = jnp.where(io == 3, 1.0, 0.0).astype(jnp.float32)

    def _chunk(w, g, k2):
        sidx_k = sidx.at[k2]
        didx_k = didx.at[k2]
        pltpu.sync_copy(xs_hbm.at[sidx_k], xs_blk)
        pltpu.sync_copy(xdb_hbm.at[didx_k], xd_blk)
        pltpu.sync_copy(ea_hbm.at[pl.ds(w * EPW + (g * GC + k2) * B, B)],
                        eam_blk)
        pltpu.sync_copy(pos_hbm.at[sidx_k], ps_blk)
        pltpu.sync_copy(pos_hbm.at[didx_k], pd_blk)

        def _edge(e, _):
            rel = ps_blk[e, pl.ds(0, L)] - pd_blk[e, pl.ds(0, L)]
            rr = rel * rel
            d2 = rr[0] + rr[1] + rr[2]
            if with_pos:
                cwacc = jnp.zeros((L,), jnp.float32)
            for j in range(H // L):
                js = pl.ds(j * L, L)
                pre = (xs_blk[e, js] + xd_blk[e, js] + eam_blk[e, js]
                       + d2 * v_regs[j])
                m = jnp.maximum(pre, 0.0)
                eam_blk[e, js] = m
                if with_pos:
                    cwacc = cwacc + m * wx_regs[j]
            if with_pos:
                cw = jnp.sum(cwacc) + bx_s
                pblk[e, :] = cw * rel + degv
            return 0

        lax.fori_loop(0, B, _edge, 0)

        pltpu.sync_copy(eam_blk, magg_acc.at[didx_k], add=True)
        if with_pos:
            pltpu.sync_copy(
                pblk, pde_hbm.at[pl.ds(w * EPW + (g * GC + k2) * B, B)])

    def _group(w, g, _):
        pltpu.sync_copy(src3d.at[w, g], sidx)
        pltpu.sync_copy(dst3d.at[w, g], didx)
        for k2 in range(GC):
            _chunk(w, g, k2)
        return 0

    lax.fori_loop(0, KG, functools.partial(_group, w), 0)
    plsc.subcore_barrier()

    for t in range(ROWS_PER_TILE // B):
        base = s * ROWS_PER_TILE + t * B
        pltpu.sync_copy(magg_acc.at[idxz.at[t]], eam_blk)
        pltpu.sync_copy(eam_blk, magg_out.at[c, pl.ds(base, B)])

    if not with_pos:
        return

    plsc.subcore_barrier()

    def _zero_row2(i, _):
        for j in range(H // L):
            xs_blk[i, pl.ds(j * L, L)] = zero16
        return 0

    lax.fori_loop(0, B, _zero_row2, 0)
    for t in range(ROWS_PER_TILE // B):
        pltpu.sync_copy(xs_blk, magg_acc.at[idxz.at[t]])
    plsc.subcore_barrier()

    def _chunk2(w, g, k2):
        didx_k = didx.at[k2]
        pltpu.sync_copy(
            pde_hbm.at[pl.ds(w * EPW + (g * GC + k2) * B, B)], pblk)

        def _copy_row(e, _):
            xs_blk[e, pl.ds(0, L)] = pblk[e, :]
            return 0

        lax.fori_loop(0, B, _copy_row, 0)
        pltpu.sync_copy(xs_blk, magg_acc.at[didx_k], add=True)

    def _group2(w, g, _):
        pltpu.sync_copy(dst3d.at[w, g], didx)
        for k2 in range(GC):
            _chunk2(w, g, k2)
        return 0

    lax.fori_loop(0, KG, functools.partial(_group2, w), 0)
    plsc.subcore_barrier()

    for t in range(ROWS_PER_TILE // B):
        base = s * ROWS_PER_TILE + t * B
        pltpu.sync_copy(magg_acc.at[idxz.at[t]], xs_blk)
        pltpu.sync_copy(xs_blk, pd_out.at[c, pl.ds(base, B)])


@functools.cache
def _make_sc_edge(with_pos):
    mesh = plsc.VectorSubcoreMesh(core_axis_name="c", subcore_axis_name="s",
                                  num_cores=NC, num_subcores=NS)
    if with_pos:
        out_type = (jax.ShapeDtypeStruct((NC, N_PAD, H), jnp.float32),
                    jax.ShapeDtypeStruct((NC, N_PAD, H), jnp.float32),
                    jax.ShapeDtypeStruct((E, L), jnp.float32))
    else:
        out_type = jax.ShapeDtypeStruct((NC, N_PAD, H), jnp.float32)
    scratch = [
        pltpu.VMEM((GC, B), jnp.int32),
        pltpu.VMEM((GC, B), jnp.int32),
        pltpu.VMEM((B, H), jnp.float32),
        pltpu.VMEM((B, H), jnp.float32),
        pltpu.VMEM((B, H), jnp.float32),
    ]
    if with_pos:
        scratch.append(pltpu.VMEM((B, L), jnp.float32))
    scratch += [
        pltpu.VMEM((B, H), jnp.float32),
        pltpu.VMEM((B, H), jnp.float32),
        pltpu.VMEM((H,), jnp.float32),
    ]
    if with_pos:
        scratch.append(pltpu.VMEM((H,), jnp.float32))
        scratch.append(pltpu.VMEM((L,), jnp.float32))
    scratch.append(pltpu.VMEM((ROWS_PER_TILE // B, B), jnp.int32))
    scratch.append(pltpu.VMEM_SHARED((N_PAD, H), jnp.float32))

    return pl.kernel(
        functools.partial(_sc_edge_body, with_pos),
        out_type=out_type,
        mesh=mesh,
        scratch_types=scratch,
        compiler_params=pltpu.CompilerParams(needs_layout_passes=False),
    )


def _sc_edge_pos(*args):
    return _make_sc_edge(True)(*args)


def _sc_edge_nopos(*args):
    return _make_sc_edge(False)(*args)


_NBLK = 2000
_EBLK = 4000


def _pre_body(x_ref, wes_ref, wed_ref, wem_ref, wc_ref, bc_ref, be_ref,
              xs_ref, xdb_ref, v_ref):
    f32 = jnp.float32
    cvec = (jnp.dot(bc_ref[...], wem_ref[...], preferred_element_type=f32)
            + be_ref[...])
    xs_ref[...] = jnp.dot(x_ref[...], wes_ref[...], preferred_element_type=f32)
    xdb_ref[...] = (jnp.dot(x_ref[...], wed_ref[...],
                            preferred_element_type=f32) + cvec)
    v_ref[...] = jnp.dot(wc_ref[...], wem_ref[...], preferred_element_type=f32)


def _tc_pre(x, wes, wed, wem, wc, bc, be):
    grid = (N // _NBLK,)
    return pl.pallas_call(
        _pre_body,
        grid=grid,
        in_specs=[
            pl.BlockSpec((_NBLK, D), lambda i: (i, 0)),
            pl.BlockSpec((D, H), lambda i: (0, 0)),
            pl.BlockSpec((D, H), lambda i: (0, 0)),
            pl.BlockSpec((H, H), lambda i: (0, 0)),
            pl.BlockSpec((1, H), lambda i: (0, 0)),
            pl.BlockSpec((1, H), lambda i: (0, 0)),
            pl.BlockSpec((1, H), lambda i: (0, 0)),
        ],
        out_specs=[
            pl.BlockSpec((_NBLK, H), lambda i: (i, 0)),
            pl.BlockSpec((_NBLK, H), lambda i: (i, 0)),
            pl.BlockSpec((1, H), lambda i: (0, 0)),
        ],
        out_shape=[
            jax.ShapeDtypeStruct((N, H), jnp.float32),
            jax.ShapeDtypeStruct((N, H), jnp.float32),
            jax.ShapeDtypeStruct((1, H), jnp.float32),
        ],
    )(x, wes, wed, wem, wc, bc, be)


def _ea_body(ea_ref, wea_ref, out_ref):
    out_ref[...] = jnp.dot(ea_ref[...], wea_ref[...],
                           preferred_element_type=jnp.float32)


def _tc_ea(edge_attr, wea):
    grid = (E // _EBLK,)
    return pl.pallas_call(
        _ea_body,
        grid=grid,
        in_specs=[
            pl.BlockSpec((_EBLK, EC), lambda i: (i, 0)),
            pl.BlockSpec((EC, H), lambda i: (0, 0)),
        ],
        out_specs=pl.BlockSpec((_EBLK, H), lambda i: (i, 0)),
        out_shape=jax.ShapeDtypeStruct((E, H), jnp.float32),
    )(edge_attr, wea)


def _post0_body(x_ref, mg_ref, pd_ref, pos_ref, wha_ref, whb_ref, bh_ref,
                xn_ref, posn_ref):
    f32 = jnp.float32
    magg = mg_ref[0] + mg_ref[1]
    pd = pd_ref[0] + pd_ref[1]
    deg = jnp.maximum(pd[:, 3:4], 1.0)
    posn_ref[...] = pos_ref[...] + pd / deg
    xn = (jnp.dot(x_ref[...], wha_ref[...], preferred_element_type=f32)
          + jnp.dot(magg, whb_ref[...], preferred_element_type=f32)
          + bh_ref[...])
    xn_ref[...] = jnp.maximum(xn, 0.0)


def _tc_post0(x, magg_p, pd_p, pospad, wha, whb, bh):
    grid = (N // _NBLK,)
    return pl.pallas_call(
        _post0_body,
        grid=grid,
        in_specs=[
            pl.BlockSpec((_NBLK, H), lambda i: (i, 0)),
            pl.BlockSpec((NC, _NBLK, H), lambda i: (0, i, 0)),
            pl.BlockSpec((NC, _NBLK, H), lambda i: (0, i, 0)),
            pl.BlockSpec((_NBLK, H), lambda i: (i, 0)),
            pl.BlockSpec((H, H), lambda i: (0, 0)),
            pl.BlockSpec((H, H), lambda i: (0, 0)),
            pl.BlockSpec((1, H), lambda i: (0, 0)),
        ],
        out_specs=[
            pl.BlockSpec((_NBLK, H), lambda i: (i, 0)),
            pl.BlockSpec((_NBLK, H), lambda i: (i, 0)),
        ],
        out_shape=[
            jax.ShapeDtypeStruct((N, H), jnp.float32),
            jax.ShapeDtypeStruct((N, H), jnp.float32),
        ],
    )(x, magg_p, pd_p, pospad, wha, whb, bh)


def _post1_body(x_ref, mg_ref, wha_ref, whb_ref, bh_ref, xn_ref):
    f32 = jnp.float32
    magg = mg_ref[0] + mg_ref[1]
    xn = (jnp.dot(x_ref[...], wha_ref[...], preferred_element_type=f32)
          + jnp.dot(magg, whb_ref[...], preferred_element_type=f32)
          + bh_ref[...])
    xn_ref[...] = jnp.maximum(xn, 0.0)


def _tc_post1(x, magg_p, wha, whb, bh):
    grid = (N // _NBLK,)
    return pl.pallas_call(
        _post1_body,
        grid=grid,
        in_specs=[
            pl.BlockSpec((_NBLK, H), lambda i: (i, 0)),
            pl.BlockSpec((NC, _NBLK, H), lambda i: (0, i, 0)),
            pl.BlockSpec((H, H), lambda i: (0, 0)),
            pl.BlockSpec((H, H), lambda i: (0, 0)),
            pl.BlockSpec((1, H), lambda i: (0, 0)),
        ],
        out_specs=pl.BlockSpec((_NBLK, H), lambda i: (i, 0)),
        out_shape=jax.ShapeDtypeStruct((N, H), jnp.float32),
    )(x, magg_p, wha, whb, bh)


def _pool_body(x_ref, b_ref, wo1_ref, bo1_ref, wo2_ref, bo2_ref, out_ref):
    f32 = jnp.float32
    giota = lax.broadcasted_iota(jnp.int32, (G, 1), 0).astype(jnp.float32)
    oh = (b_ref[...] == giota).astype(jnp.float32)
    cnt = jnp.sum(oh, axis=1, keepdims=True)
    gsum = jax.lax.dot_general(oh, x_ref[...], (((1,), (0,)), ((), ())),
                               preferred_element_type=f32)
    g = gsum / jnp.maximum(cnt, 1.0)
    h = jnp.maximum(
        jnp.dot(g, wo1_ref[...], preferred_element_type=f32) + bo1_ref[...],
        0.0)
    out_ref[...] = (jnp.dot(h, wo2_ref[...], preferred_element_type=f32)
                    + bo2_ref[...])


def _tc_pool(x2, batchf, wo1, bo1, wo2, bo2):
    return pl.pallas_call(
        _pool_body,
        out_shape=jax.ShapeDtypeStruct((G, H), jnp.float32),
    )(x2, batchf, wo1, bo1, wo2, bo2)


def kernel(x, pos, edge_index, edge_attr, batch, Wc0, bc0, We0, be0, Wx0,
           bx0, Wh0, bh0, Wc1, bc1, We1, be1, Wx1, bx1, Wh1, bh1, Wo1, bo1,
           Wo2, bo2):
    f32 = jnp.float32
    src3d = edge_index[0].reshape(NW, KG, GC, B)
    dst3d = edge_index[1].reshape(NW, KG, GC, B)
    pospad = jnp.pad(pos, ((0, 0), (0, H - 3)))

    def r2(b):
        return b.reshape(1, -1)

    xs0, xdb0, v0 = _tc_pre(x, We0[0:D], We0[D:2 * D], We0[2 * D:2 * D + H],
                            Wc0, r2(bc0), r2(be0))
    ea0 = _tc_ea(edge_attr, We0[2 * D + H:])
    iota3d = jnp.arange(N_PAD, dtype=jnp.int32).reshape(
        NS, ROWS_PER_TILE // B, B)
    magg0, pd0, _pde = _sc_edge_pos(src3d, dst3d, xs0, xdb0, ea0, pospad,
                                    v0.reshape(H), Wx0.reshape(H),
                                    jnp.full((L,), bx0[0], f32), iota3d)
    x1, posn1 = _tc_post0(x, magg0, pd0, pospad, Wh0[0:D], Wh0[D:], r2(bh0))

    xs1, xdb1, v1 = _tc_pre(x1, We1[0:H], We1[H:2 * H], We1[2 * H:3 * H],
                            Wc1, r2(bc1), r2(be1))
    ea1 = _tc_ea(edge_attr, We1[3 * H:])
    magg1 = _sc_edge_nopos(src3d, dst3d, xs1, xdb1, ea1, posn1,
                           v1.reshape(H), iota3d)
    x2 = _tc_post1(x1, magg1, Wh1[0:H], Wh1[H:], r2(bh1))

    batchf = batch.astype(f32).reshape(1, N)
    return _tc_pool(x2, batchf, Wo1, r2(bo1), Wo2, r2(bo2))


# speedup vs baseline: 3.8611x; 1.5764x over previous
import functools

import jax
import jax.numpy as jnp
from jax import lax
from jax.experimental import pallas as pl
from jax.experimental.pallas import tpu as pltpu
from jax.experimental.pallas import tpu_sc as plsc

N = 10000
E = 160000
D = 128
H = 128
EC = 16
G = 64

NC = 2
NS = 16
NW = NC * NS
L = 16

B = 40
EPW = E // NW
KPW = EPW // B
GC = 5
KG = KPW // GC
N_PAD = 10240
ROWS_PER_TILE = N_PAD // NS


def _sc_edge_body(with_pos, *refs):
    if with_pos:
        (src3d, dst3d, xs_hbm, xdb_hbm, ea_hbm, pos_hbm,
         v_hbm, wx_hbm, bx_hbm, iota_hbm,
         magg_out, pd_out, pde_hbm,
         sidx, didx, xs_blk, xd_blk, eam_blk, pblk, ps_blk, pd_blk,
         v_vm, wx_vm, bx_vm, idxz, magg_acc, sem) = refs
    else:
        (src3d, dst3d, xs_hbm, xdb_hbm, ea_hbm, pos_hbm,
         v_hbm, iota_hbm,
         magg_out,
         sidx, didx, xs_blk, xd_blk, eam_blk, ps_blk, pd_blk,
         v_vm, idxz, magg_acc, sem) = refs

    c = lax.axis_index("c")
    s = lax.axis_index("s")
    w = s * NC + c

    pltpu.sync_copy(v_hbm, v_vm)
    if with_pos:
        pltpu.sync_copy(wx_hbm, wx_vm)
        pltpu.sync_copy(bx_hbm, bx_vm)

    zero16 = jnp.zeros((L,), jnp.float32)

    def _zero_row(i, _):
        for j in range(H // L):
            eam_blk[i, pl.ds(j * L, L)] = zero16
        return 0

    lax.fori_loop(0, B, _zero_row, 0)
    pltpu.sync_copy(iota_hbm.at[s], idxz)
    for t in range(ROWS_PER_TILE // B):
        pltpu.sync_copy(eam_blk, magg_acc.at[idxz.at[t]])
    plsc.subcore_barrier()

    v_regs = [v_vm[pl.ds(j * L, L)] for j in range(H // L)]
    if with_pos:
        wx_regs = [wx_vm[pl.ds(j * L, L)] for j in range(H // L)]
        bx_s = bx_vm[...][0]
        io = lax.iota(jnp.int32, L)
        degv = jnp.where(io == 3, 1.0, 0.0).astype(jnp.float32)

    def _chunk(w, g, k2):
        sidx_k = sidx.at[k2]
        didx_k = didx.at[k2]
        cps = [
            pltpu.async_copy(xs_hbm.at[sidx_k], xs_blk, sem),
            pltpu.async_copy(xdb_hbm.at[didx_k], xd_blk, sem),
            pltpu.async_copy(
                ea_hbm.at[pl.ds(w * EPW + (g * GC + k2) * B, B)],
                eam_blk, sem),
            pltpu.async_copy(pos_hbm.at[sidx_k], ps_blk, sem),
            pltpu.async_copy(pos_hbm.at[didx_k], pd_blk, sem),
        ]
        for cp in cps:
            cp.wait()

        def _edge(e, _):
            rel = ps_blk[e, pl.ds(0, L)] - pd_blk[e, pl.ds(0, L)]
            rr = rel * rel
            d2 = rr[0] + rr[1] + rr[2]
            if with_pos:
                cwacc = jnp.zeros((L,), jnp.float32)
            for j in range(H // L):
                js = pl.ds(j * L, L)
                pre = (xs_blk[e, js] + xd_blk[e, js] + eam_blk[e, js]
                       + d2 * v_regs[j])
                m = jnp.maximum(pre, 0.0)
                eam_blk[e, js] = m
                if with_pos:
                    cwacc = cwacc + m * wx_regs[j]
            if with_pos:
                cw = jnp.sum(cwacc) + bx_s
                pblk[e, :] = cw * rel + degv
            return 0

        lax.fori_loop(0, B, _edge, 0)

        pltpu.sync_copy(eam_blk, magg_acc.at[didx_k], add=True)
        if with_pos:
            pltpu.sync_copy(
                pblk, pde_hbm.at[pl.ds(w * EPW + (g * GC + k2) * B, B)])

    def _group(w, g, _):
        c1 = pltpu.async_copy(src3d.at[w, g], sidx, sem)
        c2 = pltpu.async_copy(dst3d.at[w, g], didx, sem)
        c1.wait()
        c2.wait()
        for k2 in range(GC):
            _chunk(w, g, k2)
        return 0

    lax.fori_loop(0, KG, functools.partial(_group, w), 0)
    plsc.subcore_barrier()

    for t in range(ROWS_PER_TILE // B):
        base = s * ROWS_PER_TILE + t * B
        pltpu.sync_copy(magg_acc.at[idxz.at[t]], eam_blk)
        pltpu.sync_copy(eam_blk, magg_out.at[c, pl.ds(base, B)])

    if not with_pos:
        return

    plsc.subcore_barrier()

    def _zero_row2(i, _):
        for j in range(H // L):
            xs_blk[i, pl.ds(j * L, L)] = zero16
        return 0

    lax.fori_loop(0, B, _zero_row2, 0)
    for t in range(ROWS_PER_TILE // B):
        pltpu.sync_copy(xs_blk, magg_acc.at[idxz.at[t]])
    plsc.subcore_barrier()

    def _chunk2(w, g, k2):
        didx_k = didx.at[k2]
        pltpu.sync_copy(
            pde_hbm.at[pl.ds(w * EPW + (g * GC + k2) * B, B)], pblk)

        def _copy_row(e, _):
            xs_blk[e, pl.ds(0, L)] = pblk[e, :]
            return 0

        lax.fori_loop(0, B, _copy_row, 0)
        pltpu.sync_copy(xs_blk, magg_acc.at[didx_k], add=True)

    def _group2(w, g, _):
        pltpu.sync_copy(dst3d.at[w, g], didx)
        for k2 in range(GC):
            _chunk2(w, g, k2)
        return 0

    lax.fori_loop(0, KG, functools.partial(_group2, w), 0)
    plsc.subcore_barrier()

    for t in range(ROWS_PER_TILE // B):
        base = s * ROWS_PER_TILE + t * B
        pltpu.sync_copy(magg_acc.at[idxz.at[t]], xs_blk)
        pltpu.sync_copy(xs_blk, pd_out.at[c, pl.ds(base, B)])


@functools.cache
def _make_sc_edge(with_pos):
    mesh = plsc.VectorSubcoreMesh(core_axis_name="c", subcore_axis_name="s",
                                  num_cores=NC, num_subcores=NS)
    if with_pos:
        out_type = (jax.ShapeDtypeStruct((NC, N_PAD, H), jnp.float32),
                    jax.ShapeDtypeStruct((NC, N_PAD, H), jnp.float32),
                    jax.ShapeDtypeStruct((E, L), jnp.float32))
    else:
        out_type = jax.ShapeDtypeStruct((NC, N_PAD, H), jnp.float32)
    scratch = [
        pltpu.VMEM((GC, B), jnp.int32),
        pltpu.VMEM((GC, B), jnp.int32),
        pltpu.VMEM((B, H), jnp.float32),
        pltpu.VMEM((B, H), jnp.float32),
        pltpu.VMEM((B, H), jnp.float32),
    ]
    if with_pos:
        scratch.append(pltpu.VMEM((B, L), jnp.float32))
    scratch += [
        pltpu.VMEM((B, H), jnp.float32),
        pltpu.VMEM((B, H), jnp.float32),
        pltpu.VMEM((H,), jnp.float32),
    ]
    if with_pos:
        scratch.append(pltpu.VMEM((H,), jnp.float32))
        scratch.append(pltpu.VMEM((L,), jnp.float32))
    scratch.append(pltpu.VMEM((ROWS_PER_TILE // B, B), jnp.int32))
    scratch.append(pltpu.VMEM_SHARED((N_PAD, H), jnp.float32))
    scratch.append(pltpu.SemaphoreType.DMA)

    return pl.kernel(
        functools.partial(_sc_edge_body, with_pos),
        out_type=out_type,
        mesh=mesh,
        scratch_types=scratch,
        compiler_params=pltpu.CompilerParams(needs_layout_passes=False),
    )


def _sc_edge_pos(*args):
    return _make_sc_edge(True)(*args)


def _sc_edge_nopos(*args):
    return _make_sc_edge(False)(*args)


_NBLK = 2000
_EBLK = 4000


def _pre_body(x_ref, wes_ref, wed_ref, wem_ref, wc_ref, bc_ref, be_ref,
              xs_ref, xdb_ref, v_ref):
    f32 = jnp.float32
    cvec = (jnp.dot(bc_ref[...], wem_ref[...], preferred_element_type=f32)
            + be_ref[...])
    xs_ref[...] = jnp.dot(x_ref[...], wes_ref[...], preferred_element_type=f32)
    xdb_ref[...] = (jnp.dot(x_ref[...], wed_ref[...],
                            preferred_element_type=f32) + cvec)
    v_ref[...] = jnp.dot(wc_ref[...], wem_ref[...], preferred_element_type=f32)


def _tc_pre(x, wes, wed, wem, wc, bc, be):
    grid = (N // _NBLK,)
    return pl.pallas_call(
        _pre_body,
        grid=grid,
        in_specs=[
            pl.BlockSpec((_NBLK, D), lambda i: (i, 0)),
            pl.BlockSpec((D, H), lambda i: (0, 0)),
            pl.BlockSpec((D, H), lambda i: (0, 0)),
            pl.BlockSpec((H, H), lambda i: (0, 0)),
            pl.BlockSpec((1, H), lambda i: (0, 0)),
            pl.BlockSpec((1, H), lambda i: (0, 0)),
            pl.BlockSpec((1, H), lambda i: (0, 0)),
        ],
        out_specs=[
            pl.BlockSpec((_NBLK, H), lambda i: (i, 0)),
            pl.BlockSpec((_NBLK, H), lambda i: (i, 0)),
            pl.BlockSpec((1, H), lambda i: (0, 0)),
        ],
        out_shape=[
            jax.ShapeDtypeStruct((N, H), jnp.float32),
            jax.ShapeDtypeStruct((N, H), jnp.float32),
            jax.ShapeDtypeStruct((1, H), jnp.float32),
        ],
    )(x, wes, wed, wem, wc, bc, be)


def _ea_body(ea_ref, wea_ref, out_ref):
    out_ref[...] = jnp.dot(ea_ref[...], wea_ref[...],
                           preferred_element_type=jnp.float32)


def _tc_ea(edge_attr, wea):
    grid = (E // _EBLK,)
    return pl.pallas_call(
        _ea_body,
        grid=grid,
        in_specs=[
            pl.BlockSpec((_EBLK, EC), lambda i: (i, 0)),
            pl.BlockSpec((EC, H), lambda i: (0, 0)),
        ],
        out_specs=pl.BlockSpec((_EBLK, H), lambda i: (i, 0)),
        out_shape=jax.ShapeDtypeStruct((E, H), jnp.float32),
    )(edge_attr, wea)


def _post0_body(x_ref, mg_ref, pd_ref, pos_ref, wha_ref, whb_ref, bh_ref,
                xn_ref, posn_ref):
    f32 = jnp.float32
    magg = mg_ref[0] + mg_ref[1]
    pd = pd_ref[0] + pd_ref[1]
    deg = jnp.maximum(pd[:, 3:4], 1.0)
    posn_ref[...] = pos_ref[...] + pd / deg
    xn = (jnp.dot(x_ref[...], wha_ref[...], preferred_element_type=f32)
          + jnp.dot(magg, whb_ref[...], preferred_element_type=f32)
          + bh_ref[...])
    xn_ref[...] = jnp.maximum(xn, 0.0)


def _tc_post0(x, magg_p, pd_p, pospad, wha, whb, bh):
    grid = (N // _NBLK,)
    return pl.pallas_call(
        _post0_body,
        grid=grid,
        in_specs=[
            pl.BlockSpec((_NBLK, H), lambda i: (i, 0)),
            pl.BlockSpec((NC, _NBLK, H), lambda i: (0, i, 0)),
            pl.BlockSpec((NC, _NBLK, H), lambda i: (0, i, 0)),
            pl.BlockSpec((_NBLK, H), lambda i: (i, 0)),
            pl.BlockSpec((H, H), lambda i: (0, 0)),
            pl.BlockSpec((H, H), lambda i: (0, 0)),
            pl.BlockSpec((1, H), lambda i: (0, 0)),
        ],
        out_specs=[
            pl.BlockSpec((_NBLK, H), lambda i: (i, 0)),
            pl.BlockSpec((_NBLK, H), lambda i: (i, 0)),
        ],
        out_shape=[
            jax.ShapeDtypeStruct((N, H), jnp.float32),
            jax.ShapeDtypeStruct((N, H), jnp.float32),
        ],
    )(x, magg_p, pd_p, pospad, wha, whb, bh)


def _post1_body(x_ref, mg_ref, wha_ref, whb_ref, bh_ref, xn_ref):
    f32 = jnp.float32
    magg = mg_ref[0] + mg_ref[1]
    xn = (jnp.dot(x_ref[...], wha_ref[...], preferred_element_type=f32)
          + jnp.dot(magg, whb_ref[...], preferred_element_type=f32)
          + bh_ref[...])
    xn_ref[...] = jnp.maximum(xn, 0.0)


def _tc_post1(x, magg_p, wha, whb, bh):
    grid = (N // _NBLK,)
    return pl.pallas_call(
        _post1_body,
        grid=grid,
        in_specs=[
            pl.BlockSpec((_NBLK, H), lambda i: (i, 0)),
            pl.BlockSpec((NC, _NBLK, H), lambda i: (0, i, 0)),
            pl.BlockSpec((H, H), lambda i: (0, 0)),
            pl.BlockSpec((H, H), lambda i: (0, 0)),
            pl.BlockSpec((1, H), lambda i: (0, 0)),
        ],
        out_specs=pl.BlockSpec((_NBLK, H), lambda i: (i, 0)),
        out_shape=jax.ShapeDtypeStruct((N, H), jnp.float32),
    )(x, magg_p, wha, whb, bh)


def _pool_body(x_ref, b_ref, wo1_ref, bo1_ref, wo2_ref, bo2_ref, out_ref):
    f32 = jnp.float32
    giota = lax.broadcasted_iota(jnp.int32, (G, 1), 0).astype(jnp.float32)
    oh = (b_ref[...] == giota).astype(jnp.float32)
    cnt = jnp.sum(oh, axis=1, keepdims=True)
    gsum = jax.lax.dot_general(oh, x_ref[...], (((1,), (0,)), ((), ())),
                               preferred_element_type=f32)
    g = gsum / jnp.maximum(cnt, 1.0)
    h = jnp.maximum(
        jnp.dot(g, wo1_ref[...], preferred_element_type=f32) + bo1_ref[...],
        0.0)
    out_ref[...] = (jnp.dot(h, wo2_ref[...], preferred_element_type=f32)
                    + bo2_ref[...])


def _tc_pool(x2, batchf, wo1, bo1, wo2, bo2):
    return pl.pallas_call(
        _pool_body,
        out_shape=jax.ShapeDtypeStruct((G, H), jnp.float32),
    )(x2, batchf, wo1, bo1, wo2, bo2)


def kernel(x, pos, edge_index, edge_attr, batch, Wc0, bc0, We0, be0, Wx0,
           bx0, Wh0, bh0, Wc1, bc1, We1, be1, Wx1, bx1, Wh1, bh1, Wo1, bo1,
           Wo2, bo2):
    f32 = jnp.float32
    src3d = edge_index[0].reshape(NW, KG, GC, B)
    dst3d = edge_index[1].reshape(NW, KG, GC, B)
    pospad = jnp.pad(pos, ((0, 0), (0, H - 3)))

    def r2(b):
        return b.reshape(1, -1)

    xs0, xdb0, v0 = _tc_pre(x, We0[0:D], We0[D:2 * D], We0[2 * D:2 * D + H],
                            Wc0, r2(bc0), r2(be0))
    ea0 = _tc_ea(edge_attr, We0[2 * D + H:])
    iota3d = jnp.arange(N_PAD, dtype=jnp.int32).reshape(
        NS, ROWS_PER_TILE // B, B)
    magg0, pd0, _pde = _sc_edge_pos(src3d, dst3d, xs0, xdb0, ea0, pospad,
                                    v0.reshape(H), Wx0.reshape(H),
                                    jnp.full((L,), bx0[0], f32), iota3d)
    x1, posn1 = _tc_post0(x, magg0, pd0, pospad, Wh0[0:D], Wh0[D:], r2(bh0))

    xs1, xdb1, v1 = _tc_pre(x1, We1[0:H], We1[H:2 * H], We1[2 * H:3 * H],
                            Wc1, r2(bc1), r2(be1))
    ea1 = _tc_ea(edge_attr, We1[3 * H:])
    magg1 = _sc_edge_nopos(src3d, dst3d, xs1, xdb1, ea1, posn1,
                           v1.reshape(H), iota3d)
    x2 = _tc_post1(x1, magg1, Wh1[0:H], Wh1[H:], r2(bh1))

    batchf = batch.astype(f32).reshape(1, N)
    return _tc_pool(x2, batchf, Wo1, r2(bo1), Wo2, r2(bo2))
